# baseline (speedup 1.0000x reference)
import functools

import jax
import jax.numpy as jnp
from jax import lax
from jax.experimental import pallas as pl
from jax.experimental.pallas import tpu as pltpu
from jax.experimental.pallas import tpu_sc as plsc

N = 10000
E = 320000
D = 128

NC = 2
NS = 16
NW = NC * NS
LANES = 16

EP = E // NW
CH = 125
NCHT = E // CH
NCH_TILE = EP // CH
NPAIR = NCH_TILE // 2
NP = 10240
RPT = NP // NS
ZR = 16
NZCOPY = RPT // ZR


def _fill_2d(ref, rows, cols, value):
    v = jnp.full((LANES,), value, dtype=jnp.float32)
    per_row = cols // LANES

    def body(i, _):
        r = i // per_row
        c = (i % per_row) * LANES
        ref[r, pl.ds(c, LANES)] = v
        return 0

    lax.fori_loop(0, rows * per_row, body, 0)


def _make_sc_agg(with_deg: bool):
    out_type = [jax.ShapeDtypeStruct((NC, NP, D), jnp.float32)]
    scratch = [
        pltpu.VMEM_SHARED((NP, D), jnp.float32),
        pltpu.VMEM((2, CH), jnp.int32),
        pltpu.VMEM((2, CH), jnp.int32),
        pltpu.VMEM((CH, D), jnp.float32),
        pltpu.VMEM((CH, D), jnp.float32),
        pltpu.VMEM((ZR, D), jnp.float32),
        pltpu.SemaphoreType.DMA,
        pltpu.SemaphoreType.DMA,
    ]
    if with_deg:
        out_type.append(jax.ShapeDtypeStruct((NC, NP, LANES), jnp.float32))
        scratch += [
            pltpu.VMEM_SHARED((NP, LANES), jnp.float32),
            pltpu.VMEM((CH, LANES), jnp.float32),
            pltpu.VMEM((ZR, LANES), jnp.float32),
        ]

    mesh = plsc.VectorSubcoreMesh(
        core_axis_name="c", subcore_axis_name="s",
        num_cores=NC, num_subcores=NS)

    def body(h_hbm, ei_hbm, *refs):
        if with_deg:
            (part_hbm, deg_hbm, acc, ibuf0, ibuf1, rows0, rows1, zbuf,
             sem0, sem1, dacc, ones_v, zdbuf) = refs
        else:
            (part_hbm, acc, ibuf0, ibuf1, rows0, rows1, zbuf,
             sem0, sem1) = refs

        cid = lax.axis_index("c")
        sid = lax.axis_index("s")
        wid = cid * NS + sid

        _fill_2d(zbuf, ZR, D, 0.0)
        for k in range(NZCOPY):
            pltpu.sync_copy(zbuf, acc.at[pl.ds(sid * RPT + k * ZR, ZR)])
        if with_deg:
            _fill_2d(ones_v, CH, LANES, 1.0)
            _fill_2d(zdbuf, ZR, LANES, 0.0)
            for k in range(NZCOPY):
                pltpu.sync_copy(zdbuf, dacc.at[pl.ds(sid * RPT + k * ZR, ZR)])

        plsc.subcore_barrier()

        base = wid * NCH_TILE

        def scatter(ibuf, rows):
            pltpu.make_async_copy(h_hbm.at[ibuf.at[0]], rows, _sem(ibuf)).wait()
            pltpu.sync_copy(rows, acc.at[ibuf.at[1]], add=True)
            if with_deg:
                pltpu.sync_copy(ones_v, dacc.at[ibuf.at[1]], add=True)

        def _sem(ibuf):
            return sem0 if ibuf is ibuf0 else sem1

        def fetch(g, ibuf, rows):
            pltpu.sync_copy(ei_hbm.at[0, g], ibuf.at[0])
            pltpu.sync_copy(ei_hbm.at[1, g], ibuf.at[1])
            pltpu.async_copy(h_hbm.at[ibuf.at[0]], rows, _sem(ibuf))

        fetch(base, ibuf0, rows0)

        def pair(k, _):
            fetch(base + 2 * k + 1, ibuf1, rows1)
            scatter(ibuf0, rows0)

            @pl.when(k < NPAIR - 1)
            def _():
                fetch(base + 2 * k + 2, ibuf0, rows0)

            scatter(ibuf1, rows1)
            return 0

        lax.fori_loop(0, NPAIR, pair, 0)

        plsc.subcore_barrier()

        for k in range(NZCOPY):
            rows = pl.ds(sid * RPT + k * ZR, ZR)
            pltpu.sync_copy(acc.at[rows], part_hbm.at[cid, rows])
        if with_deg:
            rows = pl.ds(sid * RPT, RPT)
            pltpu.sync_copy(dacc.at[rows], deg_hbm.at[cid, rows])

    return pl.kernel(
        body, out_type=tuple(out_type), mesh=mesh, scratch_types=scratch,
        compiler_params=pltpu.CompilerParams(use_tc_tiling_on_sc=False))


_sc_cache = {}


def _sc_agg_fn(with_deg: bool):
    if with_deg not in _sc_cache:
        _sc_cache[with_deg] = _make_sc_agg(with_deg)
    return _sc_cache[with_deg]


BLK = 1000


def _self_body(h_ref, ws_ref, b_ref, o_ref):
    o_ref[...] = (
        jnp.dot(h_ref[...], ws_ref[...], preferred_element_type=jnp.float32)
        + b_ref[...])


_self_matmul = pl.pallas_call(
    _self_body,
    grid=(N // BLK,),
    in_specs=[
        pl.BlockSpec((BLK, D), lambda i: (i, 0)),
        pl.BlockSpec((D, D), lambda i: (0, 0)),
        pl.BlockSpec((1, D), lambda i: (0, 0)),
    ],
    out_specs=pl.BlockSpec((BLK, D), lambda i: (i, 0)),
    out_shape=jax.ShapeDtypeStruct((N, D), jnp.float32),
)


def _combine_body(relu, s_ref, p_ref, d_ref, wn_ref, o_ref):
    agg = p_ref[0] + p_ref[1]
    deg = d_ref[0, :, 0:1] + d_ref[1, :, 0:1]
    hn = agg * (1.0 / jnp.maximum(deg, 1.0))
    out = (s_ref[...]
           + jnp.dot(hn, wn_ref[...], preferred_element_type=jnp.float32))
    if relu:
        out = jnp.maximum(out, 0.0)
    o_ref[...] = out


def _make_combine(relu: bool):
    grid = (N // BLK,)
    return pl.pallas_call(
        functools.partial(_combine_body, relu),
        grid=grid,
        in_specs=[
            pl.BlockSpec((BLK, D), lambda i: (i, 0)),
            pl.BlockSpec((NC, BLK, D), lambda i: (0, i, 0)),
            pl.BlockSpec((NC, BLK, LANES), lambda i: (0, i, 0)),
            pl.BlockSpec((D, D), lambda i: (0, 0)),
        ],
        out_specs=pl.BlockSpec((BLK, D), lambda i: (i, 0)),
        out_shape=jax.ShapeDtypeStruct((N, D), jnp.float32),
    )


_combine_relu = _make_combine(True)
_combine_lin = _make_combine(False)


def kernel(h, e, edge_index, W_self0, W_neigh0, b0, W_self1, W_neigh1, b1):
    ei = edge_index.reshape(2, NCHT, CH)
    b0r = b0.reshape(1, D)
    b1r = b1.reshape(1, D)

    part0, degp = _sc_agg_fn(True)(h, ei)
    self0 = _self_matmul(h, W_self0, b0r)
    h1 = _combine_relu(self0, part0, degp, W_neigh0)
    (part1,) = _sc_agg_fn(False)(h1, ei)
    self1 = _self_matmul(h1, W_self1, b1r)
    h2 = _combine_lin(self1, part1, degp, W_neigh1)
    return (h2, e)


# speedup vs baseline: 1.2750x; 1.2750x over previous
import functools

import jax
import jax.numpy as jnp
from jax import lax
from jax.experimental import pallas as pl
from jax.experimental.pallas import tpu as pltpu
from jax.experimental.pallas import tpu_sc as plsc

N = 10000
E = 320000
D = 128

NC = 2
NS = 16
NW = NC * NS
LANES = 16

EP = E // NW
CH = 125
NCHT = E // CH
NCH_TILE = EP // CH
NPAIR = NCH_TILE // 2
NP = 10240
RPT = NP // NS
ZR = 16
NZCOPY = RPT // ZR


def _fill_2d(ref, rows, cols, value):
    v = jnp.full((LANES,), value, dtype=jnp.float32)
    per_row = cols // LANES

    def body(i, _):
        r = i // per_row
        c = (i % per_row) * LANES
        ref[r, pl.ds(c, LANES)] = v
        return 0

    lax.fori_loop(0, rows * per_row, body, 0)


NQUAD = NCH_TILE // 4


def _make_sc_agg(with_deg: bool):
    out_type = [jax.ShapeDtypeStruct((NC, NP, D), jnp.float32)]
    scratch = [
        pltpu.VMEM_SHARED((NP, D), jnp.float32),
        pltpu.VMEM((2, CH), jnp.int32),
        pltpu.VMEM((2, CH), jnp.int32),
        pltpu.VMEM((2, CH), jnp.int32),
        pltpu.VMEM((2, CH), jnp.int32),
        pltpu.VMEM((CH, D), jnp.float32),
        pltpu.VMEM((CH, D), jnp.float32),
        pltpu.VMEM((ZR, D), jnp.float32),
    ] + [pltpu.SemaphoreType.DMA] * 10
    if with_deg:
        out_type.append(jax.ShapeDtypeStruct((NC, NP, LANES), jnp.float32))
        scratch += [
            pltpu.VMEM_SHARED((NP, LANES), jnp.float32),
            pltpu.VMEM((CH, LANES), jnp.float32),
            pltpu.VMEM((ZR, LANES), jnp.float32),
        ]

    mesh = plsc.VectorSubcoreMesh(
        core_axis_name="c", subcore_axis_name="s",
        num_cores=NC, num_subcores=NS)

    def body(h_hbm, ei_hbm, *refs):
        if with_deg:
            (part_hbm, deg_hbm, acc, ib0, ib1, ib2, ib3, rows0, rows1, zbuf,
             is0a, is0b, is1a, is1b, is2a, is2b, is3a, is3b, gs0, gs1,
             dacc, ones_v, zdbuf) = refs
        else:
            (part_hbm, acc, ib0, ib1, ib2, ib3, rows0, rows1, zbuf,
             is0a, is0b, is1a, is1b, is2a, is2b, is3a, is3b, gs0, gs1) = refs

        ibufs = (ib0, ib1, ib2, ib3)
        isems = ((is0a, is0b), (is1a, is1b), (is2a, is2b), (is3a, is3b))
        rows = (rows0, rows1)
        gsems = (gs0, gs1)

        cid = lax.axis_index("c")
        sid = lax.axis_index("s")
        wid = cid * NS + sid

        _fill_2d(zbuf, ZR, D, 0.0)
        if with_deg:
            _fill_2d(ones_v, CH, LANES, 1.0)
            _fill_2d(zdbuf, ZR, LANES, 0.0)
        zcopies = []
        for k in range(NZCOPY):
            zcopies.append(
                (zbuf, acc.at[pl.ds(sid * RPT + k * ZR, ZR)], isems[k % 4][0]))
        if with_deg:
            for k in range(NZCOPY):
                zcopies.append(
                    (zdbuf, dacc.at[pl.ds(sid * RPT + k * ZR, ZR)],
                     isems[k % 4][1]))
        for s, d, sem in zcopies:
            pltpu.async_copy(s, d, sem)
        for s, d, sem in zcopies:
            pltpu.make_async_copy(s, d, sem).wait()

        plsc.subcore_barrier()

        base = wid * NCH_TILE

        def idx_fetch(cc, j):
            pltpu.async_copy(ei_hbm.at[0, cc], ibufs[j].at[0], isems[j][0])
            pltpu.async_copy(ei_hbm.at[1, cc], ibufs[j].at[1], isems[j][1])

        def idx_wait(cc, j):
            pltpu.make_async_copy(
                ei_hbm.at[0, cc], ibufs[j].at[0], isems[j][0]).wait()
            pltpu.make_async_copy(
                ei_hbm.at[1, cc], ibufs[j].at[1], isems[j][1]).wait()

        def gather_start(j, r):
            pltpu.async_copy(h_hbm.at[ibufs[j].at[0]], rows[r], gsems[r])

        def gather_wait(j, r):
            pltpu.make_async_copy(
                h_hbm.at[ibufs[j].at[0]], rows[r], gsems[r]).wait()

        idx_fetch(base + 0, 0)
        idx_fetch(base + 1, 1)
        idx_fetch(base + 2, 2)
        idx_wait(base + 0, 0)
        gather_start(0, 0)

        def quad(k, _):
            c0 = 4 * k
            for j in range(4):
                c = c0 + j
                cc = base + c
                nj = (j + 1) % 4
                fj = (j + 3) % 4

                gather_wait(j, j % 2)

                if j + 1 < 4:
                    idx_wait(cc + 1, nj)
                    gather_start(nj, (j + 1) % 2)
                else:
                    @pl.when(k < NQUAD - 1)
                    def _():
                        idx_wait(cc + 1, nj)
                        gather_start(nj, (j + 1) % 2)

                if j == 0:
                    idx_fetch(cc + 3, fj)
                else:
                    @pl.when(k < NQUAD - 1)
                    def _():
                        idx_fetch(cc + 3, fj)

                pltpu.sync_copy(rows[j % 2], acc.at[ibufs[j].at[1]], add=True)
                if with_deg:
                    pltpu.sync_copy(ones_v, dacc.at[ibufs[j].at[1]], add=True)
            return 0

        lax.fori_loop(0, NQUAD, quad, 0)

        plsc.subcore_barrier()

        sl = pl.ds(sid * RPT, RPT)
        pltpu.async_copy(acc.at[sl], part_hbm.at[cid, sl], gs0)
        if with_deg:
            pltpu.async_copy(dacc.at[sl], deg_hbm.at[cid, sl], gs1)
        pltpu.make_async_copy(acc.at[sl], part_hbm.at[cid, sl], gs0).wait()
        if with_deg:
            pltpu.make_async_copy(dacc.at[sl], deg_hbm.at[cid, sl], gs1).wait()

    return pl.kernel(
        body, out_type=tuple(out_type), mesh=mesh, scratch_types=scratch,
        compiler_params=pltpu.CompilerParams(use_tc_tiling_on_sc=False))


_sc_cache = {}


def _sc_agg_fn(with_deg: bool):
    if with_deg not in _sc_cache:
        _sc_cache[with_deg] = _make_sc_agg(with_deg)
    return _sc_cache[with_deg]


BLK = 1000


def _combine_body(relu, h_ref, p_ref, d_ref, ws_ref, wn_ref, b_ref, o_ref):
    agg = p_ref[0] + p_ref[1]
    deg = d_ref[0, :, 0:1] + d_ref[1, :, 0:1]
    hn = agg * (1.0 / jnp.maximum(deg, 1.0))
    out = (jnp.dot(h_ref[...], ws_ref[...], preferred_element_type=jnp.float32)
           + jnp.dot(hn, wn_ref[...], preferred_element_type=jnp.float32)
           + b_ref[...])
    if relu:
        out = jnp.maximum(out, 0.0)
    o_ref[...] = out


def _make_combine(relu: bool):
    grid = (N // BLK,)
    return pl.pallas_call(
        functools.partial(_combine_body, relu),
        grid=grid,
        in_specs=[
            pl.BlockSpec((BLK, D), lambda i: (i, 0)),
            pl.BlockSpec((NC, BLK, D), lambda i: (0, i, 0)),
            pl.BlockSpec((NC, BLK, LANES), lambda i: (0, i, 0)),
            pl.BlockSpec((D, D), lambda i: (0, 0)),
            pl.BlockSpec((D, D), lambda i: (0, 0)),
            pl.BlockSpec((1, D), lambda i: (0, 0)),
        ],
        out_specs=pl.BlockSpec((BLK, D), lambda i: (i, 0)),
        out_shape=jax.ShapeDtypeStruct((N, D), jnp.float32),
    )


_combine_relu = _make_combine(True)
_combine_lin = _make_combine(False)


def kernel(h, e, edge_index, W_self0, W_neigh0, b0, W_self1, W_neigh1, b1):
    ei = edge_index.reshape(2, NCHT, CH)
    b0r = b0.reshape(1, D)
    b1r = b1.reshape(1, D)

    part0, degp = _sc_agg_fn(True)(h, ei)
    h1 = _combine_relu(h, part0, degp, W_self0, W_neigh0, b0r)
    (part1,) = _sc_agg_fn(False)(h1, ei)
    h2 = _combine_lin(h1, part1, degp, W_self1, W_neigh1, b1r)
    return (h2, e)


# speedup vs baseline: 1.2822x; 1.0056x over previous
import functools

import jax
import jax.numpy as jnp
from jax import lax
from jax.experimental import pallas as pl
from jax.experimental.pallas import tpu as pltpu
from jax.experimental.pallas import tpu_sc as plsc

N = 10000
E = 320000
D = 128

NC = 2
NS = 16
NW = NC * NS
LANES = 16

EP = E // NW
CH = 125
NCHT = E // CH
NCH_TILE = EP // CH
NPAIR = NCH_TILE // 2
NP = 10240
RPT = NP // NS
ZR = 16
NZCOPY = RPT // ZR


def _fill_2d(ref, rows, cols, value):
    v = jnp.full((LANES,), value, dtype=jnp.float32)
    per_row = cols // LANES

    def body(i, _):
        r = i // per_row
        c = (i % per_row) * LANES
        ref[r, pl.ds(c, LANES)] = v
        return 0

    lax.fori_loop(0, rows * per_row, body, 0)


NQUAD = NCH_TILE // 4


def _make_sc_agg(with_deg: bool):
    out_type = [jax.ShapeDtypeStruct((NC, NP, D), jnp.float32)]
    scratch = [
        pltpu.VMEM_SHARED((NP, D), jnp.float32),
        pltpu.VMEM((2, CH), jnp.int32),
        pltpu.VMEM((2, CH), jnp.int32),
        pltpu.VMEM((2, CH), jnp.int32),
        pltpu.VMEM((2, CH), jnp.int32),
        pltpu.VMEM((CH, D), jnp.float32),
        pltpu.VMEM((CH, D), jnp.float32),
        pltpu.VMEM((ZR, D), jnp.float32),
    ] + [pltpu.SemaphoreType.DMA] * 10
    if with_deg:
        out_type.append(jax.ShapeDtypeStruct((NC, NP, LANES), jnp.float32))
        scratch += [
            pltpu.VMEM_SHARED((NP, LANES), jnp.float32),
            pltpu.VMEM((CH, LANES), jnp.float32),
            pltpu.VMEM((ZR, LANES), jnp.float32),
        ]

    mesh = plsc.VectorSubcoreMesh(
        core_axis_name="c", subcore_axis_name="s",
        num_cores=NC, num_subcores=NS)

    def body(h_hbm, ei_hbm, *refs):
        if with_deg:
            (part_hbm, deg_hbm, acc, ib0, ib1, ib2, ib3, rows0, rows1, zbuf,
             is0a, is0b, is1a, is1b, is2a, is2b, is3a, is3b, gs0, gs1,
             dacc, ones_v, zdbuf) = refs
        else:
            (part_hbm, acc, ib0, ib1, ib2, ib3, rows0, rows1, zbuf,
             is0a, is0b, is1a, is1b, is2a, is2b, is3a, is3b, gs0, gs1) = refs

        ibufs = (ib0, ib1, ib2, ib3)
        isems = ((is0a, is0b), (is1a, is1b), (is2a, is2b), (is3a, is3b))
        rows = (rows0, rows1)
        gsems = (gs0, gs1)

        cid = lax.axis_index("c")
        sid = lax.axis_index("s")
        wid = cid * NS + sid

        _fill_2d(zbuf, ZR, D, 0.0)
        if with_deg:
            _fill_2d(ones_v, CH, LANES, 1.0)
            _fill_2d(zdbuf, ZR, LANES, 0.0)
        zcopies = []
        for k in range(NZCOPY):
            zcopies.append(
                (zbuf, acc.at[pl.ds(sid * RPT + k * ZR, ZR)], isems[k % 4][0]))
        if with_deg:
            for k in range(NZCOPY):
                zcopies.append(
                    (zdbuf, dacc.at[pl.ds(sid * RPT + k * ZR, ZR)],
                     isems[k % 4][1]))
        for s, d, sem in zcopies:
            pltpu.async_copy(s, d, sem)
        for s, d, sem in zcopies:
            pltpu.make_async_copy(s, d, sem).wait()

        plsc.subcore_barrier()

        base = wid * NCH_TILE

        def idx_fetch(cc, j):
            pltpu.async_copy(ei_hbm.at[0, cc], ibufs[j].at[0], isems[j][0])
            pltpu.async_copy(ei_hbm.at[1, cc], ibufs[j].at[1], isems[j][1])

        def idx_wait(cc, j):
            pltpu.make_async_copy(
                ei_hbm.at[0, cc], ibufs[j].at[0], isems[j][0]).wait()
            pltpu.make_async_copy(
                ei_hbm.at[1, cc], ibufs[j].at[1], isems[j][1]).wait()

        def gather_start(j, r):
            pltpu.async_copy(h_hbm.at[ibufs[j].at[0]], rows[r], gsems[r])

        def gather_wait(j, r):
            pltpu.make_async_copy(
                h_hbm.at[ibufs[j].at[0]], rows[r], gsems[r]).wait()

        idx_fetch(base + 0, 0)
        idx_fetch(base + 1, 1)
        idx_fetch(base + 2, 2)
        idx_wait(base + 0, 0)
        gather_start(0, 0)

        def quad(k, _):
            c0 = 4 * k
            for j in range(4):
                c = c0 + j
                cc = base + c
                nj = (j + 1) % 4
                fj = (j + 3) % 4

                gather_wait(j, j % 2)

                if j + 1 < 4:
                    idx_wait(cc + 1, nj)
                    gather_start(nj, (j + 1) % 2)
                else:
                    @pl.when(k < NQUAD - 1)
                    def _():
                        idx_wait(cc + 1, nj)
                        gather_start(nj, (j + 1) % 2)

                if j == 0:
                    idx_fetch(cc + 3, fj)
                else:
                    @pl.when(k < NQUAD - 1)
                    def _():
                        idx_fetch(cc + 3, fj)

                pltpu.sync_copy(rows[j % 2], acc.at[ibufs[j].at[1]], add=True)
                if with_deg:
                    pltpu.sync_copy(ones_v, dacc.at[ibufs[j].at[1]], add=True)
            return 0

        lax.fori_loop(0, NQUAD, quad, 0)

        plsc.subcore_barrier()

        sl = pl.ds(sid * RPT, RPT)
        pltpu.async_copy(acc.at[sl], part_hbm.at[cid, sl], gs0)
        if with_deg:
            pltpu.async_copy(dacc.at[sl], deg_hbm.at[cid, sl], gs1)
        pltpu.make_async_copy(acc.at[sl], part_hbm.at[cid, sl], gs0).wait()
        if with_deg:
            pltpu.make_async_copy(dacc.at[sl], deg_hbm.at[cid, sl], gs1).wait()

    return pl.kernel(
        body, out_type=tuple(out_type), mesh=mesh, scratch_types=scratch,
        compiler_params=pltpu.CompilerParams(use_tc_tiling_on_sc=False))


_sc_cache = {}


def _sc_agg_fn(with_deg: bool):
    if with_deg not in _sc_cache:
        _sc_cache[with_deg] = _make_sc_agg(with_deg)
    return _sc_cache[with_deg]


BLK = 1000


def _selfmm_body(h_ref, ws_ref, b_ref, o_ref):
    o_ref[...] = (
        jnp.dot(h_ref[...], ws_ref[...], preferred_element_type=jnp.float32)
        + b_ref[...])


_selfmm = pl.pallas_call(
    _selfmm_body,
    grid=(N // BLK,),
    in_specs=[
        pl.BlockSpec((BLK, D), lambda i: (i, 0)),
        pl.BlockSpec((D, D), lambda i: (0, 0)),
        pl.BlockSpec((1, D), lambda i: (0, 0)),
    ],
    out_specs=pl.BlockSpec((BLK, D), lambda i: (i, 0)),
    out_shape=jax.ShapeDtypeStruct((N, D), jnp.float32),
)


def _combine_body(relu, s_ref, p_ref, d_ref, wn_ref, o_ref):
    agg = p_ref[0] + p_ref[1]
    deg = d_ref[0, :, 0:1] + d_ref[1, :, 0:1]
    hn = agg * (1.0 / jnp.maximum(deg, 1.0))
    out = s_ref[...] + jnp.dot(hn, wn_ref[...],
                               preferred_element_type=jnp.float32)
    if relu:
        out = jnp.maximum(out, 0.0)
    o_ref[...] = out


def _make_combine(relu: bool):
    grid = (N // BLK,)
    return pl.pallas_call(
        functools.partial(_combine_body, relu),
        grid=grid,
        in_specs=[
            pl.BlockSpec((BLK, D), lambda i: (i, 0)),
            pl.BlockSpec((NC, BLK, D), lambda i: (0, i, 0)),
            pl.BlockSpec((NC, BLK, LANES), lambda i: (0, i, 0)),
            pl.BlockSpec((D, D), lambda i: (0, 0)),
        ],
        out_specs=pl.BlockSpec((BLK, D), lambda i: (i, 0)),
        out_shape=jax.ShapeDtypeStruct((N, D), jnp.float32),
    )


_combine_relu = _make_combine(True)
_combine_lin = _make_combine(False)


def kernel(h, e, edge_index, W_self0, W_neigh0, b0, W_self1, W_neigh1, b1):
    ei = edge_index.reshape(2, NCHT, CH)
    b0r = b0.reshape(1, D)
    b1r = b1.reshape(1, D)

    part0, degp = _sc_agg_fn(True)(h, ei)
    smm0 = _selfmm(h, W_self0, b0r)
    h1 = _combine_relu(smm0, part0, degp, W_neigh0)
    (part1,) = _sc_agg_fn(False)(h1, ei)
    smm1 = _selfmm(h1, W_self1, b1r)
    h2 = _combine_lin(smm1, part1, degp, W_neigh1)
    return (h2, e)
